# baseline (speedup 1.0000x reference)
import functools

import jax
import jax.numpy as jnp
from jax import lax
from jax.experimental import pallas as pl
from jax.experimental.pallas import tpu as pltpu
from jax.experimental.pallas import tpu_sc as plsc

HIDDEN = 768
LANES = 16
HB = HIDDEN // LANES
NUM_CORES = 2
NUM_SUBCORES = 16
NUM_WORKERS = NUM_CORES * NUM_SUBCORES
CHUNK = 32


def _sc_lookup(n_tokens):
    tpw = n_tokens // NUM_WORKERS
    n_chunks = tpw // CHUNK
    assert n_chunks % 2 == 0

    mesh = plsc.VectorSubcoreMesh(core_axis_name="c", subcore_axis_name="s")

    @functools.partial(
        pl.kernel,
        out_type=(
            jax.ShapeDtypeStruct((n_tokens, HIDDEN), jnp.float32),
            jax.ShapeDtypeStruct((n_tokens, HIDDEN), jnp.float32),
        ),
        mesh=mesh,
        scratch_types=[
            pltpu.VMEM((5, HIDDEN), jnp.float32),
            pltpu.VMEM((5, HIDDEN), jnp.float32),
            pltpu.VMEM((tpw + LANES,), jnp.int32),
            pltpu.VMEM((tpw + LANES,), jnp.float32),
            pltpu.VMEM((2 * CHUNK, HIDDEN), jnp.float32),
            pltpu.VMEM((2 * CHUNK, HIDDEN), jnp.float32),
            pltpu.SemaphoreType.DMA,
            pltpu.SemaphoreType.DMA,
            pltpu.SemaphoreType.DMA,
            pltpu.SemaphoreType.DMA,
        ],
    )
    def body(rp_hbm, mask_hbm, ktab_hbm, vtab_hbm, outk_hbm, outv_hbm,
             ktab, vtab, idx, msk, kbuf, vbuf, ks0, ks1, vs0, vs1):
        ksems = (ks0, ks1)
        vsems = (vs0, vs1)
        wid = lax.axis_index("s") * NUM_CORES + lax.axis_index("c")
        base = wid * tpw
        pltpu.sync_copy(ktab_hbm, ktab)
        pltpu.sync_copy(vtab_hbm, vtab)
        pltpu.sync_copy(rp_hbm.at[pl.ds(base, tpw)], idx.at[pl.ds(0, tpw)])
        pltpu.sync_copy(mask_hbm.at[pl.ds(base, tpw)],
                        msk.at[pl.ds(0, tpw)])

        def drain(par):
            pltpu.make_async_copy(
                kbuf.at[pl.ds(par * CHUNK, CHUNK)],
                outk_hbm.at[pl.ds(base, CHUNK)], ksems[par]).wait()
            pltpu.make_async_copy(
                vbuf.at[pl.ds(par * CHUNK, CHUNK)],
                outv_hbm.at[pl.ds(base, CHUNK)], vsems[par]).wait()

        def chunk_body(c, carry):
            p = lax.rem(c, 2)

            @pl.when(c >= 2)
            def _():
                @pl.when(p == 0)
                def _():
                    drain(0)

                @pl.when(p == 1)
                def _():
                    drain(1)

            def tok_body(t, carry2):
                tok = c * CHUNK + t
                s = idx[pl.ds(tok, LANES)][0]
                m = msk[pl.ds(tok, LANES)][0]
                row = p * CHUNK + t
                for k in range(HB):
                    sl = pl.ds(k * LANES, LANES)
                    kbuf[row, sl] = ktab[s, sl] * m
                    vbuf[row, sl] = vtab[s, sl] * m
                return carry2

            lax.fori_loop(0, CHUNK, tok_body, 0, unroll=False)

            row0 = base + c * CHUNK

            @pl.when(p == 0)
            def _():
                pltpu.async_copy(kbuf.at[pl.ds(0, CHUNK)],
                                 outk_hbm.at[pl.ds(row0, CHUNK)], ks0)
                pltpu.async_copy(vbuf.at[pl.ds(0, CHUNK)],
                                 outv_hbm.at[pl.ds(row0, CHUNK)], vs0)

            @pl.when(p == 1)
            def _():
                pltpu.async_copy(kbuf.at[pl.ds(CHUNK, CHUNK)],
                                 outk_hbm.at[pl.ds(row0, CHUNK)], ks1)
                pltpu.async_copy(vbuf.at[pl.ds(CHUNK, CHUNK)],
                                 outv_hbm.at[pl.ds(row0, CHUNK)], vs1)

            return carry

        lax.fori_loop(0, n_chunks, chunk_body, 0, unroll=False)
        drain(0)
        drain(1)

    return body


def kernel(relative_positions, entity_mask, entity_pos_key_table,
           entity_pos_value_table):
    b, s = relative_positions.shape
    n = b * s
    rp = relative_positions.reshape(n).astype(jnp.int32)
    msk = entity_mask.reshape(n)
    out_k, out_v = _sc_lookup(n)(rp, msk, entity_pos_key_table,
                                 entity_pos_value_table)
    h = entity_pos_key_table.shape[1]
    return out_k.reshape(b, s, h), out_v.reshape(b, s, h)


# speedup vs baseline: 3.0324x; 3.0324x over previous
import functools

import jax
import jax.numpy as jnp
from jax import lax
from jax.experimental import pallas as pl
from jax.experimental.pallas import tpu as pltpu
from jax.experimental.pallas import tpu_sc as plsc

HIDDEN = 768
LANES = 16
HB = HIDDEN // LANES
NUM_CORES = 2
NUM_SUBCORES = 16
NUM_WORKERS = NUM_CORES * NUM_SUBCORES
CHUNK = 32


def _sc_lookup(n_tokens):
    tpw = n_tokens // NUM_WORKERS
    n_chunks = tpw // CHUNK
    assert n_chunks % 2 == 0

    mesh = plsc.VectorSubcoreMesh(core_axis_name="c", subcore_axis_name="s")

    @functools.partial(
        pl.kernel,
        out_type=(
            jax.ShapeDtypeStruct((n_tokens, HIDDEN), jnp.float32),
            jax.ShapeDtypeStruct((n_tokens, HIDDEN), jnp.float32),
        ),
        mesh=mesh,
        scratch_types=[
            pltpu.VMEM((5, HIDDEN), jnp.float32),
            pltpu.VMEM((5, HIDDEN), jnp.float32),
            pltpu.VMEM((tpw + LANES,), jnp.int32),
            pltpu.VMEM((tpw + LANES,), jnp.float32),
            pltpu.VMEM((2 * CHUNK, HIDDEN), jnp.float32),
            pltpu.VMEM((2 * CHUNK, HIDDEN), jnp.float32),
            pltpu.SemaphoreType.DMA,
            pltpu.SemaphoreType.DMA,
            pltpu.SemaphoreType.DMA,
            pltpu.SemaphoreType.DMA,
        ],
    )
    def body(rp_hbm, mask_hbm, ktab_hbm, vtab_hbm, outk_hbm, outv_hbm,
             ktab, vtab, idx, msk, kbuf, vbuf, ks0, ks1, vs0, vs1):
        ksems = (ks0, ks1)
        vsems = (vs0, vs1)
        wid = lax.axis_index("s") * NUM_CORES + lax.axis_index("c")
        base = wid * tpw
        pltpu.sync_copy(ktab_hbm, ktab)
        pltpu.sync_copy(vtab_hbm, vtab)
        pltpu.sync_copy(rp_hbm.at[pl.ds(base, tpw)], idx.at[pl.ds(0, tpw)])
        pltpu.sync_copy(mask_hbm.at[pl.ds(base, tpw)],
                        msk.at[pl.ds(0, tpw)])

        def drain(par):
            pltpu.make_async_copy(
                kbuf.at[pl.ds(par * CHUNK, CHUNK)],
                outk_hbm.at[pl.ds(base, CHUNK)], ksems[par]).wait()
            pltpu.make_async_copy(
                vbuf.at[pl.ds(par * CHUNK, CHUNK)],
                outv_hbm.at[pl.ds(base, CHUNK)], vsems[par]).wait()

        def chunk_body(c, carry):
            p = lax.rem(c, 2)

            @pl.when(c >= 2)
            def _():
                @pl.when(p == 0)
                def _():
                    drain(0)

                @pl.when(p == 1)
                def _():
                    drain(1)

            @plsc.parallel_loop(0, CHUNK, unroll=2)
            def tok_body(t):
                tok = c * CHUNK + t
                s = idx[pl.ds(tok, LANES)][0]
                m = msk[pl.ds(tok, LANES)][0]
                row = p * CHUNK + t
                for k in range(HB):
                    sl = pl.ds(k * LANES, LANES)
                    kbuf[row, sl] = ktab[s, sl] * m
                    vbuf[row, sl] = vtab[s, sl] * m

            row0 = base + c * CHUNK

            @pl.when(p == 0)
            def _():
                pltpu.async_copy(kbuf.at[pl.ds(0, CHUNK)],
                                 outk_hbm.at[pl.ds(row0, CHUNK)], ks0)
                pltpu.async_copy(vbuf.at[pl.ds(0, CHUNK)],
                                 outv_hbm.at[pl.ds(row0, CHUNK)], vs0)

            @pl.when(p == 1)
            def _():
                pltpu.async_copy(kbuf.at[pl.ds(CHUNK, CHUNK)],
                                 outk_hbm.at[pl.ds(row0, CHUNK)], ks1)
                pltpu.async_copy(vbuf.at[pl.ds(CHUNK, CHUNK)],
                                 outv_hbm.at[pl.ds(row0, CHUNK)], vs1)

            return carry

        lax.fori_loop(0, n_chunks, chunk_body, 0, unroll=False)
        drain(0)
        drain(1)

    return body


def kernel(relative_positions, entity_mask, entity_pos_key_table,
           entity_pos_value_table):
    b, s = relative_positions.shape
    n = b * s
    rp = relative_positions.reshape(n).astype(jnp.int32)
    msk = entity_mask.reshape(n)
    out_k, out_v = _sc_lookup(n)(rp, msk, entity_pos_key_table,
                                 entity_pos_value_table)
    h = entity_pos_key_table.shape[1]
    return out_k.reshape(b, s, h), out_v.reshape(b, s, h)
